# baseline (speedup 1.0000x reference)
import jax
import jax.numpy as jnp
from jax import lax
from jax.experimental import pallas as pl
from jax.experimental.pallas import tpu as pltpu
from jax.experimental.pallas import tpu_sc as plsc

N_NODES_K = 10000
N_PAD = 10240
D_K = 128
NNZ_K = 320000

NUM_CORES = 2
NUM_SUBCORES = 16
NUM_WORKERS = NUM_CORES * NUM_SUBCORES
EDGES_PER_WORKER = NNZ_K // NUM_WORKERS
BLK = 80
NBLK = EDGES_PER_WORKER // BLK
ROWS_PER_TILE = N_PAD // NUM_SUBCORES


def _hop_body(with_combine, t_hbm, cols_hbm, rows_hbm, vals_hbm, out_hbm,
              tcomb_hbm,
              cols0, cols1, cols2, cols3, rows0, rows1, rows2, rows3,
              vals0, vals1, vals2, vals3, gbuf0, gbuf1, gbuf2, gbuf3,
              acc,
              semg0, semg1, semg2, semg3, sems0, sems1, sems2, sems3,
              semv0, semv1, semv2, semv3, semb0, semb1, semb2, semb3,
              semr0, semr1, semr2, semr3):
    c = lax.axis_index("c")
    s = lax.axis_index("s")
    w = s * NUM_CORES + c
    ebase = w * EDGES_PER_WORKER

    cols = (cols0, cols1, cols2, cols3)
    rows = (rows0, rows1, rows2, rows3)
    vals = (vals0, vals1, vals2, vals3)
    gbuf = (gbuf0, gbuf1, gbuf2, gbuf3)
    semg = (semg0, semg1, semg2, semg3)
    sems = (sems0, sems1, sems2, sems3)
    semv = (semv0, semv1, semv2, semv3)
    semb = (semb0, semb1, semb2, semb3)
    semr = (semr0, semr1, semr2, semr3)

    if with_combine:
        gsrc = tcomb_hbm
        for k in range(ROWS_PER_TILE // BLK):
            off = s * ROWS_PER_TILE + k * BLK
            pltpu.sync_copy(t_hbm.at[0, pl.ds(off, BLK)], gbuf0)
            pltpu.sync_copy(t_hbm.at[1, pl.ds(off, BLK)], gbuf1)

            def add_chunk(i, _):
                sl = (i // 8, pl.ds((i % 8) * 16, 16))
                gbuf0[sl] = gbuf0[sl] + gbuf1[sl]
                return 0

            lax.fori_loop(0, BLK * 8, add_chunk, 0)
            pltpu.sync_copy(gbuf0, tcomb_hbm.at[pl.ds(c * N_PAD + off, BLK)])
    else:
        gsrc = t_hbm

    def bias_cols(buf):
        if with_combine:
            base = c * N_PAD
            for i in range(BLK // 16):
                buf[pl.ds(i * 16, 16)] = buf[pl.ds(i * 16, 16)] + base

    def zero_chunk(i, _):
        gbuf0[i // 8, pl.ds((i % 8) * 16, 16)] = jnp.zeros((16,), jnp.float32)
        return 0

    lax.fori_loop(0, BLK * 8, zero_chunk, 0)
    for k in range(ROWS_PER_TILE // BLK):
        pltpu.sync_copy(gbuf0, acc.at[pl.ds(s * ROWS_PER_TILE + k * BLK, BLK)])

    plsc.subcore_barrier()

    def blk_issue(m, buf2, sem2, hbm):
        off = ebase + jnp.minimum(m, NBLK - 1) * BLK
        pltpu.async_copy(hbm.at[pl.ds(off, BLK)], buf2, sem2)

    def blk_wait(buf2, sem2, hbm):
        pltpu.make_async_copy(hbm.at[pl.ds(0, BLK)], buf2, sem2).wait()

    def scatter_wait(x):
        pltpu.make_async_copy(gbuf[x], acc.at[rows[x]], sems[x]).wait()

    def process(n, x, swait, g2, vlast):
        z = (x + 2) % 4
        pltpu.make_async_copy(gsrc.at[cols[x]], gbuf[x], semg[x]).wait()
        if g2:
            blk_issue(n + 4, cols[x], semb[x], cols_hbm)
        if swait:
            scatter_wait(z)
            if g2:
                blk_issue(n + 2, rows[z], semr[z], rows_hbm)
        if g2:
            blk_wait(cols[z], semb[z], cols_hbm)
            bias_cols(cols[z])
            pltpu.async_copy(gsrc.at[cols[z]], gbuf[z], semg[z])
        blk_wait(vals[x], semv[x], vals_hbm)

        def scale_group(eb, _):
            val16 = vals[x][pl.ds(eb * 16, 16)]
            for l in range(16):
                v = val16[l]
                e = eb * 16 + l
                for j in range(8):
                    sl = (e, pl.ds(j * 16, 16))
                    gbuf[x][sl] = gbuf[x][sl] * v
            return 0

        lax.fori_loop(0, BLK // 16, scale_group, 0)
        blk_wait(rows[x], semr[x], rows_hbm)
        pltpu.async_copy(gbuf[x], acc.at[rows[x]], sems[x], add=True)
        if not vlast:
            blk_issue(n + 4, vals[x], semv[x], vals_hbm)

    for i in range(4):
        blk_issue(i, cols[i], semb[i], cols_hbm)
        blk_issue(i, rows[i], semr[i], rows_hbm)
        blk_issue(i, vals[i], semv[i], vals_hbm)
    blk_wait(cols0, semb0, cols_hbm)
    bias_cols(cols0)
    pltpu.async_copy(gsrc.at[cols0], gbuf0, semg0)
    blk_wait(cols1, semb1, cols_hbm)
    bias_cols(cols1)
    pltpu.async_copy(gsrc.at[cols1], gbuf1, semg1)

    process(0, 0, False, True, False)
    process(1, 1, False, True, False)

    def quad_body(p, _):
        process(4 * p + 2, 2, True, True, False)
        process(4 * p + 3, 3, True, True, False)
        process(4 * p + 4, 0, True, True, False)
        process(4 * p + 5, 1, True, True, False)
        return 0

    lax.fori_loop(0, (NBLK - 5) // 4, quad_body, 0)
    process(NBLK - 3, 2, True, True, False)
    process(NBLK - 2, 3, True, False, False)
    process(NBLK - 1, 0, True, False, True)
    scatter_wait(3)
    scatter_wait(0)
    for x in (1, 2, 3):
        blk_wait(vals[x], semv[x], vals_hbm)
    for x in (1, 2):
        blk_wait(cols[x], semb[x], cols_hbm)

    plsc.subcore_barrier()

    for k in range(ROWS_PER_TILE // BLK):
        off = s * ROWS_PER_TILE + k * BLK
        pltpu.sync_copy(acc.at[pl.ds(off, BLK)], out_hbm.at[c, pl.ds(off, BLK)])


def _hop_body_first(t, cols_h, rows_h, vals_h, out_h, *scr):
    _hop_body(False, t, cols_h, rows_h, vals_h, out_h, None, *scr)


def _hop_body_comb(pp, cols_h, rows_h, vals_h, out_h, tcomb_h, *scr):
    _hop_body(True, pp, cols_h, rows_h, vals_h, out_h, tcomb_h, *scr)


_SCRATCH = (
    [pltpu.VMEM((BLK,), jnp.int32)] * 4
    + [pltpu.VMEM((BLK,), jnp.int32)] * 4
    + [pltpu.VMEM((BLK,), jnp.float32)] * 4
    + [pltpu.VMEM((BLK, D_K), jnp.float32)] * 4
    + [pltpu.VMEM_SHARED((N_PAD, D_K), jnp.float32)]
    + [pltpu.SemaphoreType.DMA] * 20
)
_PART = jax.ShapeDtypeStruct((NUM_CORES, N_PAD, D_K), jnp.float32)
_TCOMB = jax.ShapeDtypeStruct((NUM_CORES * N_PAD, D_K), jnp.float32)


@jax.jit
def _hop_first(t, cols1d, rows1d, vals):
    mesh = plsc.VectorSubcoreMesh(core_axis_name="c", subcore_axis_name="s")
    f = pl.kernel(_hop_body_first, mesh=mesh, out_type=_PART,
                  scratch_types=list(_SCRATCH))
    return f(t, cols1d, rows1d, vals)


@jax.jit
def _hop_comb(pp, cols1d, rows1d, vals):
    mesh = plsc.VectorSubcoreMesh(core_axis_name="c", subcore_axis_name="s")
    f = pl.kernel(_hop_body_comb, mesh=mesh, out_type=[_PART, _TCOMB],
                  scratch_types=list(_SCRATCH))
    return f(pp, cols1d, rows1d, vals)


def _add_body(a_ref, b_ref, o_ref):
    o_ref[...] = a_ref[...] + b_ref[...]


@jax.jit
def _combine(p):
    spec = pl.BlockSpec((1024, D_K), lambda i: (i, 0))
    return pl.pallas_call(
        _add_body,
        grid=(N_PAD // 1024,),
        in_specs=[spec, spec],
        out_specs=spec,
        out_shape=jax.ShapeDtypeStruct((N_PAD, D_K), jnp.float32),
    )(p[0], p[1])


def kernel(user_embed, item_embed, adj_rows, adj_cols, adj_vals):
    t0 = jnp.concatenate(
        [user_embed, item_embed,
         jnp.zeros((N_PAD - N_NODES_K, D_K), jnp.float32)], axis=0)

    p = _hop_first(t0, adj_cols, adj_rows, adj_vals)
    p, t1c = _hop_comb(p, adj_cols, adj_rows, adj_vals)
    p, t2c = _hop_comb(p, adj_cols, adj_rows, adj_vals)
    t3 = _combine(p)
    embs = [t0, t1c[:N_PAD], t2c[:N_PAD], t3]
    embs = jnp.stack(embs, axis=1)
    n_users = user_embed.shape[0]
    return embs[:n_users], embs[n_users:N_NODES_K]


# speedup vs baseline: 1.1502x; 1.1502x over previous
import jax
import jax.numpy as jnp
from jax import lax
from jax.experimental import pallas as pl
from jax.experimental.pallas import tpu as pltpu
from jax.experimental.pallas import tpu_sc as plsc

N_NODES_K = 10000
N_PAD = 10240
D_K = 128
NNZ_K = 320000

NUM_CORES = 2
NUM_SUBCORES = 16
NUM_WORKERS = NUM_CORES * NUM_SUBCORES
EDGES_PER_WORKER = NNZ_K // NUM_WORKERS
BLK = 80
NBLK = EDGES_PER_WORKER // BLK
ROWS_PER_TILE = N_PAD // NUM_SUBCORES


def _hop_body(t_hbm, cols_hbm, rows_hbm, vals_hbm, out_hbm,
              cols0, cols1, cols2, cols3, rows0, rows1, rows2, rows3,
              vals0, vals1, vals2, vals3, gbuf0, gbuf1, gbuf2, gbuf3,
              acc,
              semg0, semg1, semg2, semg3, sems0, sems1, sems2, sems3,
              semv0, semv1, semv2, semv3, semb0, semb1, semb2, semb3,
              semr0, semr1, semr2, semr3):
    c = lax.axis_index("c")
    s = lax.axis_index("s")
    w = s * NUM_CORES + c
    ebase = w * EDGES_PER_WORKER

    cols = (cols0, cols1, cols2, cols3)
    rows = (rows0, rows1, rows2, rows3)
    vals = (vals0, vals1, vals2, vals3)
    gbuf = (gbuf0, gbuf1, gbuf2, gbuf3)
    semg = (semg0, semg1, semg2, semg3)
    sems = (sems0, sems1, sems2, sems3)
    semv = (semv0, semv1, semv2, semv3)
    semb = (semb0, semb1, semb2, semb3)
    semr = (semr0, semr1, semr2, semr3)

    def zero_chunk(i, _):
        gbuf0[i // 8, pl.ds((i % 8) * 16, 16)] = jnp.zeros((16,), jnp.float32)
        return 0

    lax.fori_loop(0, BLK * 8, zero_chunk, 0)
    for k in range(ROWS_PER_TILE // BLK):
        pltpu.sync_copy(gbuf0, acc.at[pl.ds(s * ROWS_PER_TILE + k * BLK, BLK)])

    plsc.subcore_barrier()

    def blk_issue(m, buf2, sem2, hbm):
        off = ebase + jnp.minimum(m, NBLK - 1) * BLK
        pltpu.async_copy(hbm.at[pl.ds(off, BLK)], buf2, sem2)

    def blk_wait(buf2, sem2, hbm):
        pltpu.make_async_copy(hbm.at[pl.ds(0, BLK)], buf2, sem2).wait()

    def scatter_wait(x):
        pltpu.make_async_copy(gbuf[x], acc.at[rows[x]], sems[x]).wait()

    def process(n, x, swait, g2, vlast):
        z = (x + 2) % 4
        pltpu.make_async_copy(t_hbm.at[cols[x]], gbuf[x], semg[x]).wait()
        if g2:
            blk_issue(n + 4, cols[x], semb[x], cols_hbm)
        if swait:
            scatter_wait(z)
            if g2:
                blk_issue(n + 2, rows[z], semr[z], rows_hbm)
        if g2:
            blk_wait(cols[z], semb[z], cols_hbm)
            pltpu.async_copy(t_hbm.at[cols[z]], gbuf[z], semg[z])
        blk_wait(vals[x], semv[x], vals_hbm)

        def scale_group(eb, _):
            val16 = vals[x][pl.ds(eb * 16, 16)]
            for l in range(16):
                v = val16[l]
                e = eb * 16 + l
                for j in range(8):
                    sl = (e, pl.ds(j * 16, 16))
                    gbuf[x][sl] = gbuf[x][sl] * v
            return 0

        lax.fori_loop(0, BLK // 16, scale_group, 0)
        blk_wait(rows[x], semr[x], rows_hbm)
        pltpu.async_copy(gbuf[x], acc.at[rows[x]], sems[x], add=True)
        if not vlast:
            blk_issue(n + 4, vals[x], semv[x], vals_hbm)

    for i in range(4):
        blk_issue(i, cols[i], semb[i], cols_hbm)
        blk_issue(i, rows[i], semr[i], rows_hbm)
        blk_issue(i, vals[i], semv[i], vals_hbm)
    blk_wait(cols0, semb0, cols_hbm)
    pltpu.async_copy(t_hbm.at[cols0], gbuf0, semg0)
    blk_wait(cols1, semb1, cols_hbm)
    pltpu.async_copy(t_hbm.at[cols1], gbuf1, semg1)

    process(0, 0, False, True, False)
    process(1, 1, False, True, False)

    def quad_body(p, _):
        process(4 * p + 2, 2, True, True, False)
        process(4 * p + 3, 3, True, True, False)
        process(4 * p + 4, 0, True, True, False)
        process(4 * p + 5, 1, True, True, False)
        return 0

    lax.fori_loop(0, (NBLK - 5) // 4, quad_body, 0)
    process(NBLK - 3, 2, True, True, False)
    process(NBLK - 2, 3, True, False, False)
    process(NBLK - 1, 0, True, False, True)
    scatter_wait(3)
    scatter_wait(0)
    for x in (1, 2, 3):
        blk_wait(vals[x], semv[x], vals_hbm)
    for x in (1, 2):
        blk_wait(cols[x], semb[x], cols_hbm)

    plsc.subcore_barrier()

    for k in range(ROWS_PER_TILE // BLK):
        off = s * ROWS_PER_TILE + k * BLK
        pltpu.sync_copy(acc.at[pl.ds(off, BLK)], out_hbm.at[c, pl.ds(off, BLK)])


_SCRATCH = (
    [pltpu.VMEM((BLK,), jnp.int32)] * 4
    + [pltpu.VMEM((BLK,), jnp.int32)] * 4
    + [pltpu.VMEM((BLK,), jnp.float32)] * 4
    + [pltpu.VMEM((BLK, D_K), jnp.float32)] * 4
    + [pltpu.VMEM_SHARED((N_PAD, D_K), jnp.float32)]
    + [pltpu.SemaphoreType.DMA] * 20
)
_PART = jax.ShapeDtypeStruct((NUM_CORES, N_PAD, D_K), jnp.float32)


@jax.jit
def _hop(t, cols1d, rows1d, vals):
    mesh = plsc.VectorSubcoreMesh(core_axis_name="c", subcore_axis_name="s")
    f = pl.kernel(_hop_body, mesh=mesh, out_type=_PART,
                  scratch_types=list(_SCRATCH))
    return f(t, cols1d, rows1d, vals)


def _add_body(a_ref, b_ref, o_ref):
    o_ref[...] = a_ref[...] + b_ref[...]


@jax.jit
def _combine(p):
    spec = pl.BlockSpec((1024, D_K), lambda i: (i, 0))
    return pl.pallas_call(
        _add_body,
        grid=(N_PAD // 1024,),
        in_specs=[spec, spec],
        out_specs=spec,
        out_shape=jax.ShapeDtypeStruct((N_PAD, D_K), jnp.float32),
    )(p[0], p[1])


def kernel(user_embed, item_embed, adj_rows, adj_cols, adj_vals):
    t0 = jnp.concatenate(
        [user_embed, item_embed,
         jnp.zeros((N_PAD - N_NODES_K, D_K), jnp.float32)], axis=0)

    embs = [t0]
    t = t0
    for _ in range(3):
        p = _hop(t, adj_cols, adj_rows, adj_vals)
        t = _combine(p)
        embs.append(t)
    embs = jnp.stack(embs, axis=1)
    n_users = user_embed.shape[0]
    return embs[:n_users], embs[n_users:N_NODES_K]


# speedup vs baseline: 1.1513x; 1.0009x over previous
import jax
import jax.numpy as jnp
from jax import lax
from jax.experimental import pallas as pl
from jax.experimental.pallas import tpu as pltpu
from jax.experimental.pallas import tpu_sc as plsc

N_NODES_K = 10000
N_PAD = 10240
D_K = 128
NNZ_K = 320000

NUM_CORES = 2
NUM_SUBCORES = 16
NUM_WORKERS = NUM_CORES * NUM_SUBCORES
EDGES_PER_WORKER = NNZ_K // NUM_WORKERS
BLK = 80
NBLK = EDGES_PER_WORKER // BLK
ROWS_PER_TILE = N_PAD // NUM_SUBCORES


def _hop_body(t_hbm, cols_hbm, rows_hbm, vals_hbm, out_hbm,
              cols0, cols1, cols2, cols3, rows0, rows1, rows2, rows3,
              vals0, vals1, vals2, vals3, gbuf0, gbuf1, gbuf2, gbuf3,
              acc,
              semg0, semg1, semg2, semg3, sems0, sems1, sems2, sems3,
              semv0, semv1, semv2, semv3, semb0, semb1, semb2, semb3,
              semr0, semr1, semr2, semr3):
    c = lax.axis_index("c")
    s = lax.axis_index("s")
    w = s * NUM_CORES + c
    ebase = w * EDGES_PER_WORKER

    cols = (cols0, cols1, cols2, cols3)
    rows = (rows0, rows1, rows2, rows3)
    vals = (vals0, vals1, vals2, vals3)
    gbuf = (gbuf0, gbuf1, gbuf2, gbuf3)
    semg = (semg0, semg1, semg2, semg3)
    sems = (sems0, sems1, sems2, sems3)
    semv = (semv0, semv1, semv2, semv3)
    semb = (semb0, semb1, semb2, semb3)
    semr = (semr0, semr1, semr2, semr3)

    def zero_chunk(i, _):
        gbuf0[i // 8, pl.ds((i % 8) * 16, 16)] = jnp.zeros((16,), jnp.float32)
        return 0

    lax.fori_loop(0, BLK * 8, zero_chunk, 0)
    for k in range(ROWS_PER_TILE // BLK):
        pltpu.async_copy(
            gbuf0, acc.at[pl.ds(s * ROWS_PER_TILE + k * BLK, BLK)], semg0)
    for k in range(ROWS_PER_TILE // BLK):
        pltpu.make_async_copy(
            gbuf0, acc.at[pl.ds(s * ROWS_PER_TILE + k * BLK, BLK)],
            semg0).wait()

    plsc.subcore_barrier()

    def blk_issue(m, buf2, sem2, hbm):
        off = ebase + jnp.minimum(m, NBLK - 1) * BLK
        pltpu.async_copy(hbm.at[pl.ds(off, BLK)], buf2, sem2)

    def blk_wait(buf2, sem2, hbm):
        pltpu.make_async_copy(hbm.at[pl.ds(0, BLK)], buf2, sem2).wait()

    def scatter_wait(x):
        pltpu.make_async_copy(gbuf[x], acc.at[rows[x]], sems[x]).wait()

    def process(n, x, swait, g2, vlast):
        z = (x + 2) % 4
        pltpu.make_async_copy(t_hbm.at[cols[x]], gbuf[x], semg[x]).wait()
        if g2:
            blk_issue(n + 4, cols[x], semb[x], cols_hbm)
        if swait:
            scatter_wait(z)
            if g2:
                blk_issue(n + 2, rows[z], semr[z], rows_hbm)
        if g2:
            blk_wait(cols[z], semb[z], cols_hbm)
            pltpu.async_copy(t_hbm.at[cols[z]], gbuf[z], semg[z])
        blk_wait(vals[x], semv[x], vals_hbm)

        def scale_group(eb, _):
            val16 = vals[x][pl.ds(eb * 16, 16)]
            for l in range(16):
                v = val16[l]
                e = eb * 16 + l
                for j in range(8):
                    sl = (e, pl.ds(j * 16, 16))
                    gbuf[x][sl] = gbuf[x][sl] * v
            return 0

        lax.fori_loop(0, BLK // 16, scale_group, 0)
        blk_wait(rows[x], semr[x], rows_hbm)
        pltpu.async_copy(gbuf[x], acc.at[rows[x]], sems[x], add=True)
        if not vlast:
            blk_issue(n + 4, vals[x], semv[x], vals_hbm)

    for i in range(4):
        blk_issue(i, cols[i], semb[i], cols_hbm)
        blk_issue(i, rows[i], semr[i], rows_hbm)
        blk_issue(i, vals[i], semv[i], vals_hbm)
    blk_wait(cols0, semb0, cols_hbm)
    pltpu.async_copy(t_hbm.at[cols0], gbuf0, semg0)
    blk_wait(cols1, semb1, cols_hbm)
    pltpu.async_copy(t_hbm.at[cols1], gbuf1, semg1)

    process(0, 0, False, True, False)
    process(1, 1, False, True, False)

    def quad_body(p, _):
        process(4 * p + 2, 2, True, True, False)
        process(4 * p + 3, 3, True, True, False)
        process(4 * p + 4, 0, True, True, False)
        process(4 * p + 5, 1, True, True, False)
        return 0

    lax.fori_loop(0, (NBLK - 5) // 4, quad_body, 0)
    process(NBLK - 3, 2, True, True, False)
    process(NBLK - 2, 3, True, False, False)
    process(NBLK - 1, 0, True, False, True)
    scatter_wait(3)
    scatter_wait(0)
    for x in (1, 2, 3):
        blk_wait(vals[x], semv[x], vals_hbm)
    for x in (1, 2):
        blk_wait(cols[x], semb[x], cols_hbm)

    plsc.subcore_barrier()

    for k in range(ROWS_PER_TILE // BLK):
        off = s * ROWS_PER_TILE + k * BLK
        pltpu.async_copy(
            acc.at[pl.ds(off, BLK)], out_hbm.at[c, pl.ds(off, BLK)], semg0)
    for k in range(ROWS_PER_TILE // BLK):
        off = s * ROWS_PER_TILE + k * BLK
        pltpu.make_async_copy(
            acc.at[pl.ds(off, BLK)], out_hbm.at[c, pl.ds(off, BLK)],
            semg0).wait()


_SCRATCH = (
    [pltpu.VMEM((BLK,), jnp.int32)] * 4
    + [pltpu.VMEM((BLK,), jnp.int32)] * 4
    + [pltpu.VMEM((BLK,), jnp.float32)] * 4
    + [pltpu.VMEM((BLK, D_K), jnp.float32)] * 4
    + [pltpu.VMEM_SHARED((N_PAD, D_K), jnp.float32)]
    + [pltpu.SemaphoreType.DMA] * 20
)
_PART = jax.ShapeDtypeStruct((NUM_CORES, N_PAD, D_K), jnp.float32)


@jax.jit
def _hop(t, cols1d, rows1d, vals):
    mesh = plsc.VectorSubcoreMesh(core_axis_name="c", subcore_axis_name="s")
    f = pl.kernel(_hop_body, mesh=mesh, out_type=_PART,
                  scratch_types=list(_SCRATCH))
    return f(t, cols1d, rows1d, vals)


def _add_body(a_ref, b_ref, o_ref):
    o_ref[...] = a_ref[...] + b_ref[...]


@jax.jit
def _combine(p):
    spec = pl.BlockSpec((1024, D_K), lambda i: (i, 0))
    return pl.pallas_call(
        _add_body,
        grid=(N_PAD // 1024,),
        in_specs=[spec, spec],
        out_specs=spec,
        out_shape=jax.ShapeDtypeStruct((N_PAD, D_K), jnp.float32),
    )(p[0], p[1])


def kernel(user_embed, item_embed, adj_rows, adj_cols, adj_vals):
    t0 = jnp.concatenate(
        [user_embed, item_embed,
         jnp.zeros((N_PAD - N_NODES_K, D_K), jnp.float32)], axis=0)

    embs = [t0]
    t = t0
    for _ in range(3):
        p = _hop(t, adj_cols, adj_rows, adj_vals)
        t = _combine(p)
        embs.append(t)
    embs = jnp.stack(embs, axis=1)
    n_users = user_embed.shape[0]
    return embs[:n_users], embs[n_users:N_NODES_K]
